# baseline (speedup 1.0000x reference)
import jax
import jax.numpy as jnp
from jax.experimental import pallas as pl
from jax.experimental.pallas import tpu as pltpu

T = 1024
EMBED = 256
B = 128
NT = T // B
NEG = -1.0e30


def _shift_left(x, s, fill):
    pad = jnp.full(x.shape[:-1] + (s,), fill, x.dtype)
    return jnp.concatenate([x[..., s:], pad], axis=-1)


def _shift_right(x, s, fill):
    pad = jnp.full(x.shape[:-1] + (s,), fill, x.dtype)
    return jnp.concatenate([pad, x[..., : x.shape[-1] - s]], axis=-1)


def _prefix_max(x):
    n = x.shape[-1]
    s = 1
    while s < n:
        x = jnp.maximum(x, _shift_right(x, s, NEG))
        s *= 2
    return x


def _suffix_max_excl(x):
    n = x.shape[-1]
    x = _shift_left(x, 1, NEG)
    s = 1
    while s < n:
        x = jnp.maximum(x, _shift_left(x, s, NEG))
        s *= 2
    return x


def _prefix_sum_2level(a8):
    x = a8
    s = 1
    while s < B:
        x = x + _shift_right(x, s, 0.0)
        s *= 2
    row_tot = x[:, B - 1 : B]
    off = jnp.concatenate(
        [jnp.zeros((1, 1), jnp.float32), row_tot[: NT - 1, :]], axis=0
    )
    s = 1
    while s < NT:
        pad = jnp.zeros((s, 1), jnp.float32)
        off = off + jnp.concatenate([pad, off[: NT - s, :]], axis=0)
        s *= 2
    return x + off


def _stack_kernel(u_ref, d_ref, v_ref, out_ref, m_s, p_s, cs_s, rr_s, v_s):
    r = pl.program_id(0)

    @pl.when(r == 0)
    def _scans():
        a8 = d_ref[...] - u_ref[...]
        p8 = _prefix_sum_2level(a8)
        m8 = d_ref[...] - p8
        m_s[...] = m8
        p_s[...] = p8
        cs_s[...] = jnp.concatenate(
            [_suffix_max_excl(m8[c : c + 1, :]) for c in range(NT)], axis=0
        )
        rr_s[...] = jnp.concatenate(
            [_prefix_max(m8[c : c + 1, :]) for c in range(NT)], axis=0
        )

    v_bf = v_ref[...].astype(jnp.bfloat16)
    v_s[pl.ds(r * B, B), :] = v_bf

    lane = jax.lax.broadcasted_iota(jnp.int32, (B, B), 1)
    subl = jax.lax.broadcasted_iota(jnp.int32, (B, B), 0)
    lower = lane <= subl
    neg11 = jnp.full((1, 1), NEG, jnp.float32)

    for rs in range(NT):

        @pl.when(r == rs)
        def _row(rs=rs):
            m8 = m_s[...]
            ps_col = p_s[rs : rs + 1, :].reshape(B, 1)
            rr_col = rr_s[rs : rs + 1, :].reshape(B, 1)
            m_row = m8[rs : rs + 1, :]

            tile_max = [rr_s[c : c + 1, B - 1 : B] for c in range(NT)]
            mid = [neg11] * NT
            for c in range(rs - 2, -1, -1):
                mid[c] = jnp.maximum(tile_max[c + 1], mid[c + 1])

            if rs > 0:
                col_strip = cs_s[0:rs, :].reshape(1, B * rs)
                m_strip = m8[0:rs, :].reshape(1, B * rs)
                mid_strip = jnp.concatenate(
                    [jnp.broadcast_to(mid[c], (1, B)) for c in range(rs)],
                    axis=1,
                )
                g = jnp.maximum(jnp.maximum(col_strip, mid_strip), rr_col)
                ct = jnp.minimum(
                    jnp.maximum(m_strip - g, 0.0),
                    jnp.maximum(ps_col + g, 0.0),
                )
                strip = jnp.dot(
                    ct.astype(jnp.bfloat16),
                    v_s[0 : B * rs, :],
                    preferred_element_type=jnp.float32,
                )
            else:
                strip = jnp.zeros((B, EMBED), jnp.float32)

            a2 = jnp.where(lower, jnp.broadcast_to(m_row, (B, B)), NEG)
            g = _suffix_max_excl(a2)
            ct = jnp.minimum(
                jnp.maximum(m_row - g, 0.0), jnp.maximum(ps_col + g, 0.0)
            )
            out_ref[...] = strip + jnp.dot(
                ct.astype(jnp.bfloat16),
                v_bf,
                preferred_element_type=jnp.float32,
            )


@jax.jit
def kernel(v, u, d):
    u8 = u.reshape(NT, B)
    d8 = d.reshape(NT, B)
    return pl.pallas_call(
        _stack_kernel,
        grid=(NT,),
        in_specs=[
            pl.BlockSpec(memory_space=pltpu.VMEM),
            pl.BlockSpec(memory_space=pltpu.VMEM),
            pl.BlockSpec((B, EMBED), lambda r: (r, 0)),
        ],
        out_specs=pl.BlockSpec((B, EMBED), lambda r: (r, 0)),
        out_shape=jax.ShapeDtypeStruct((T, EMBED), jnp.float32),
        scratch_shapes=[
            pltpu.VMEM((NT, B), jnp.float32),
            pltpu.VMEM((NT, B), jnp.float32),
            pltpu.VMEM((NT, B), jnp.float32),
            pltpu.VMEM((NT, B), jnp.float32),
            pltpu.VMEM((T, EMBED), jnp.bfloat16),
        ],
    )(u8, d8, v)


# speedup vs baseline: 2.0614x; 2.0614x over previous
import jax
import jax.numpy as jnp
from jax.experimental import pallas as pl
from jax.experimental.pallas import tpu as pltpu

T = 1024
EMBED = 256
B = 128
NT = T // B
NEG = -1.0e30


def _shift_left(x, s, fill):
    pad = jnp.full(x.shape[:-1] + (s,), fill, x.dtype)
    return jnp.concatenate([x[..., s:], pad], axis=-1)


def _shift_right(x, s, fill):
    pad = jnp.full(x.shape[:-1] + (s,), fill, x.dtype)
    return jnp.concatenate([pad, x[..., : x.shape[-1] - s]], axis=-1)


def _prefix_max(x):
    n = x.shape[-1]
    s = 1
    while s < n:
        x = jnp.maximum(x, _shift_right(x, s, NEG))
        s *= 2
    return x


def _suffix_max_excl(x):
    n = x.shape[-1]
    x = _shift_left(x, 1, NEG)
    s = 1
    while s < n:
        x = jnp.maximum(x, _shift_left(x, s, NEG))
        s *= 2
    return x


def _prefix_sum_2level(a8):
    x = a8
    s = 1
    while s < B:
        x = x + _shift_right(x, s, 0.0)
        s *= 2
    row_tot = x[:, B - 1 : B]
    off = jnp.concatenate(
        [jnp.zeros((1, 1), jnp.float32), row_tot[: NT - 1, :]], axis=0
    )
    s = 1
    while s < NT:
        pad = jnp.zeros((s, 1), jnp.float32)
        off = off + jnp.concatenate([pad, off[: NT - s, :]], axis=0)
        s *= 2
    return x + off


def _stack_kernel(u_ref, d_ref, v_ref, out_ref):
    v_bf = v_ref[...].astype(jnp.bfloat16)
    a8 = d_ref[...] - u_ref[...]
    p8 = _prefix_sum_2level(a8)
    m8 = d_ref[...] - p8

    ms = [m8[c : c + 1, :] for c in range(NT)]
    ps = [p8[r : r + 1, :] for r in range(NT)]
    col_s = [_suffix_max_excl(ms[c]) for c in range(NT)]
    row_r = [_prefix_max(ms[r]) for r in range(NT)]
    tile_max = [row_r[c][:, B - 1 : B] for c in range(NT)]

    ps_col = [ps[r].reshape(B, 1) for r in range(NT)]
    row_r_col = [row_r[r].reshape(B, 1) for r in range(NT)]

    lane = jax.lax.broadcasted_iota(jnp.int32, (B, B), 1)
    subl = jax.lax.broadcasted_iota(jnp.int32, (B, B), 0)
    lower = lane <= subl

    neg11 = jnp.full((1, 1), NEG, jnp.float32)

    for r in range(NT):
        mid = [neg11] * NT
        for c in range(r - 2, -1, -1):
            mid[c] = jnp.maximum(tile_max[c + 1], mid[c + 1])

        if r > 0:
            col_strip = jnp.concatenate(col_s[:r], axis=1)
            m_strip = jnp.concatenate(ms[:r], axis=1)
            mid_strip = jnp.concatenate(
                [jnp.broadcast_to(mid[c], (1, B)) for c in range(r)], axis=1
            )
            g = jnp.maximum(
                jnp.maximum(col_strip, mid_strip), row_r_col[r]
            )
            ct = jnp.minimum(
                jnp.maximum(m_strip - g, 0.0),
                jnp.maximum(ps_col[r] + g, 0.0),
            )
        else:
            ct = None

        a2 = jnp.where(lower, jnp.broadcast_to(ms[r], (B, B)), NEG)
        g = _suffix_max_excl(a2)
        ct_diag = jnp.minimum(
            jnp.maximum(ms[r] - g, 0.0), jnp.maximum(ps_col[r] + g, 0.0)
        )
        if r > 0:
            ct_row = jnp.concatenate([ct, ct_diag], axis=1)
        else:
            ct_row = ct_diag
        out_ref[r * B : (r + 1) * B, :] = jnp.dot(
            ct_row.astype(jnp.bfloat16),
            v_bf[: B * (r + 1), :],
            preferred_element_type=jnp.float32,
        )


@jax.jit
def kernel(v, u, d):
    u8 = u.reshape(NT, B)
    d8 = d.reshape(NT, B)
    return pl.pallas_call(
        _stack_kernel,
        in_specs=[
            pl.BlockSpec(memory_space=pltpu.VMEM),
            pl.BlockSpec(memory_space=pltpu.VMEM),
            pl.BlockSpec(memory_space=pltpu.VMEM),
        ],
        out_specs=pl.BlockSpec(memory_space=pltpu.VMEM),
        out_shape=jax.ShapeDtypeStruct((T, EMBED), jnp.float32),
    )(u8, d8, v)
